# baseline (speedup 1.0000x reference)
import functools

import jax
import jax.numpy as jnp
from jax import lax
from jax.experimental import pallas as pl
from jax.experimental.pallas import tpu as pltpu
from jax.experimental.pallas import tpu_sc as plsc

NC = 2
NS = 16
NW = NC * NS
SUB = 80
NRING = 5

_EPS = 1e-5


def _pre_body(x_ref, wa_ref, wb_ref, xa_ref, xb_ref):
    xx = x_ref[...]
    xa_ref[...] = jnp.dot(xx, wa_ref[...], preferred_element_type=jnp.float32)
    xb_ref[...] = jnp.dot(xx, wb_ref[...], preferred_element_type=jnp.float32)


def _pre_call(x8, wa_sp, wb_sp):
    n8, dp = x8.shape
    bn = n8
    grid = (n8 // bn,)
    return pl.pallas_call(
        _pre_body,
        grid=grid,
        in_specs=[
            pl.BlockSpec((bn, dp), lambda i: (i, 0)),
            pl.BlockSpec((dp, 128), lambda i: (0, 0)),
            pl.BlockSpec((dp, 128), lambda i: (0, 0)),
        ],
        out_specs=[
            pl.BlockSpec((bn, 128), lambda i: (i, 0)),
            pl.BlockSpec((bn, 128), lambda i: (i, 0)),
        ],
        out_shape=[
            jax.ShapeDtypeStruct((n8, 128), jnp.float32),
            jax.ShapeDtypeStruct((n8, 128), jnp.float32),
        ],
    )(x8, wa_sp, wb_sp)


def _gather_call(xa, xb, ii, jj, e, de):
    ew = e // NW
    ch = ew // SUB
    assert ch % NRING == 0
    mesh = plsc.VectorSubcoreMesh(core_axis_name="c", subcore_axis_name="s")

    @functools.partial(
        pl.kernel,
        out_type=jax.ShapeDtypeStruct((de, e), jnp.float32),
        mesh=mesh,
        scratch_types=[
            pltpu.VMEM((ew,), jnp.int32),
            pltpu.VMEM((ew,), jnp.int32),
            pltpu.VMEM((NRING, SUB, de), jnp.float32),
            pltpu.VMEM((NRING, SUB, de), jnp.float32),
            pltpu.VMEM((NRING, de, SUB), jnp.float32),
            [pltpu.SemaphoreType.DMA] * NRING,
            [pltpu.SemaphoreType.DMA] * NRING,
        ],
        compiler_params=pltpu.CompilerParams(use_tc_tiling_on_sc=False, needs_layout_passes=False),
    )
    def gather_kernel(xa_hbm, xb_hbm, ii_hbm, jj_hbm, g_hbm,
                      ii_v, jj_v, ra_v, rb_v, st_v, sem_g, sem_s):
        cid = lax.axis_index("c")
        sid = lax.axis_index("s")
        wid = sid * NC + cid
        pltpu.sync_copy(ii_hbm.at[pl.ds(wid * ew, ew)], ii_v)
        pltpu.sync_copy(jj_hbm.at[pl.ds(wid * ew, ew)], jj_v)
        dummy = g_hbm.at[:, pl.ds(0, SUB)]
        dummy2 = xa_hbm.at[pl.ds(0, SUB)]
        lane = lax.iota(jnp.int32, de)

        def fire(k, u):
            idx = pl.multiple_of(k * SUB, 8)
            pltpu.async_copy(xa_hbm.at[ii_v.at[pl.ds(idx, SUB)]],
                             ra_v.at[u], sem_g[u])
            pltpu.async_copy(xb_hbm.at[jj_v.at[pl.ds(idx, SUB)]],
                             rb_v.at[u], sem_g[u])

        def drain_gather(u):
            pltpu.make_async_copy(dummy2, ra_v.at[u], sem_g[u]).wait()
            pltpu.make_async_copy(dummy2, rb_v.at[u], sem_g[u]).wait()

        fire(0, 0)
        fire(1, 1)
        fire(2, 2)

        def super_step(s5, carry):
            for u in range(NRING):
                k = s5 * NRING + u
                u3 = (u + 3) % NRING
                @pl.when(k >= 2)
                def _():
                    pltpu.make_async_copy(dummy, st_v.at[u3], sem_s[u3]).wait()

                @pl.when(k + 3 < ch)
                def _():
                    fire(k + 3, u3)

                drain_gather(u)

                def add_row(r, c2):
                    plsc.store_scatter(
                        st_v.at[u], [lane, jnp.full((de,), r, jnp.int32)],
                        ra_v[u, r, :] + rb_v[u, r, :])
                    return c2

                lax.fori_loop(0, SUB, add_row, 0, unroll=8)
                col0 = pl.multiple_of((wid * ch + k) * SUB, 8)
                pltpu.async_copy(
                    st_v.at[u], g_hbm.at[:, pl.ds(col0, SUB)], sem_s[u])
            return carry

        lax.fori_loop(0, ch // NRING, super_step, 0)
        for u in range(NRING - 2, NRING):
            pltpu.make_async_copy(dummy, st_v.at[u], sem_s[u]).wait()

    return gather_kernel(xa, xb, ii, jj)


def _edge_body(g_ref, ea_ref, w1c_ref, w2_ref, w3_ref,
               b1_ref, b2_ref, b3_ref, gam_ref, bet_ref, out_ref):
    ea = ea_ref[...]
    z = (jnp.dot(w1c_ref[...], ea, preferred_element_type=jnp.float32)
         + g_ref[...] + b1_ref[...])
    h = z * jax.nn.sigmoid(z)
    z2 = jnp.dot(w2_ref[...], h, preferred_element_type=jnp.float32) + b2_ref[...]
    h2 = z2 * jax.nn.sigmoid(z2)
    o = jnp.dot(w3_ref[...], h2, preferred_element_type=jnp.float32) + b3_ref[...]
    m = jnp.mean(o, axis=0, keepdims=True)
    dlt = o - m
    v = jnp.mean(dlt * dlt, axis=0, keepdims=True)
    ln = dlt * lax.rsqrt(v + _EPS) * gam_ref[...] + bet_ref[...]
    out_ref[...] = ea + ln


def _edge_call(g_t, ea_t, w1cT, w2T, w3T, b1c, b2c, b3c, gc, bc):
    de, e = g_t.shape
    be = 16000
    grid = (e // be,)
    blk = lambda i: (0, i)
    zro = lambda i: (0, 0)
    return pl.pallas_call(
        _edge_body,
        grid=grid,
        in_specs=[
            pl.BlockSpec((de, be), blk),
            pl.BlockSpec((de, be), blk),
            pl.BlockSpec((de, de), zro),
            pl.BlockSpec((de, de), zro),
            pl.BlockSpec((de, de), zro),
            pl.BlockSpec((de, 1), zro),
            pl.BlockSpec((de, 1), zro),
            pl.BlockSpec((de, 1), zro),
            pl.BlockSpec((de, 1), zro),
            pl.BlockSpec((de, 1), zro),
        ],
        out_specs=pl.BlockSpec((de, be), blk),
        out_shape=jax.ShapeDtypeStruct((de, e), jnp.float32),
    )(g_t, ea_t, w1cT, w2T, w3T, b1c, b2c, b3c, gc, bc)


def _scatter_call(ea2_flat, jj, zeros_nde, npad, e, de):
    ew = e // NW
    ch = ew // SUB
    assert ch % NRING == 0
    zr = npad // NS
    mesh = plsc.VectorSubcoreMesh(core_axis_name="c", subcore_axis_name="s")

    @functools.partial(
        pl.kernel,
        out_type=jax.ShapeDtypeStruct((NC, npad, de), jnp.float32),
        mesh=mesh,
        scratch_types=[
            pltpu.VMEM((ew,), jnp.int32),
            pltpu.VMEM((NRING, de, 5 * SUB), jnp.float32),
            pltpu.VMEM((SUB, de), jnp.float32),
            pltpu.VMEM_SHARED((npad, de), jnp.float32),
            [pltpu.SemaphoreType.DMA] * NRING,
        ],
        compiler_params=pltpu.CompilerParams(use_tc_tiling_on_sc=False, needs_layout_passes=False),
    )
    def scatter_kernel(ea2_hbm, jj_hbm, z_hbm, agg_hbm,
                       jj_v, rows_v, st_v, acc_sh, sem_l):
        cid = lax.axis_index("c")
        sid = lax.axis_index("s")
        wid = sid * NC + cid
        lw = 5 * SUB
        nl = ch // 5
        pltpu.sync_copy(z_hbm.at[pl.ds(sid * zr, zr)],
                        acc_sh.at[pl.ds(sid * zr, zr)])
        plsc.subcore_barrier()
        pltpu.sync_copy(jj_hbm.at[pl.ds(wid * ew, ew)], jj_v)
        dummy = ea2_hbm.at[:, pl.ds(0, 5 * SUB)]
        lane = lax.iota(jnp.int32, de)

        def fire(l, u):
            col0 = pl.multiple_of((wid * nl + l) * lw, 8)
            pltpu.async_copy(ea2_hbm.at[:, pl.ds(col0, lw)],
                             rows_v.at[u], sem_l[u])

        fire(0, 0)
        fire(1, 1)

        def super_step(s5, carry):
            for u in range(NRING):
                l = s5 * NRING + u
                u2 = (u + 2) % NRING

                @pl.when(l + 2 < nl)
                def _():
                    fire(l + 2, u2)

                pltpu.make_async_copy(dummy, rows_v.at[u], sem_l[u]).wait()
                for s in range(5):
                    k = l * 5 + s

                    def stage_row(r, c2):
                        st_v[r, :] = plsc.load_gather(
                            rows_v.at[u],
                            [lane, jnp.full((de,), s * SUB + r, jnp.int32)])
                        return c2

                    lax.fori_loop(0, SUB, stage_row, 0, unroll=8)
                    idx = pl.multiple_of(k * SUB, 8)
                    pltpu.sync_copy(st_v,
                                    acc_sh.at[jj_v.at[pl.ds(idx, SUB)]],
                                    add=True)
            return carry

        lax.fori_loop(0, nl // NRING, super_step, 0)
        plsc.subcore_barrier()
        pltpu.sync_copy(acc_sh.at[pl.ds(sid * zr, zr)],
                        agg_hbm.at[cid].at[pl.ds(sid * zr, zr)])

    return scatter_kernel(ea2_flat, jj, zeros_nde)


def _node_body(x_ref, a0_ref, a1_ref, w1a_ref, w1b_ref, w2_ref, w3_ref,
               b1_ref, b2_ref, b3_ref, gam_ref, bet_ref, out_ref):
    xx = x_ref[...]
    agg = a0_ref[...] + a1_ref[...]
    z = (jnp.dot(xx, w1a_ref[...], preferred_element_type=jnp.float32)
         + jnp.dot(agg, w1b_ref[...], preferred_element_type=jnp.float32)
         + b1_ref[...])
    h = z * jax.nn.sigmoid(z)
    z2 = jnp.dot(h, w2_ref[...], preferred_element_type=jnp.float32) + b2_ref[...]
    h2 = z2 * jax.nn.sigmoid(z2)
    o = jnp.dot(h2, w3_ref[...], preferred_element_type=jnp.float32) + b3_ref[...]
    m = jnp.mean(o, axis=-1, keepdims=True)
    dlt = o - m
    v = jnp.mean(dlt * dlt, axis=-1, keepdims=True)
    out_ref[...] = xx + dlt * lax.rsqrt(v + _EPS) * gam_ref[...] + bet_ref[...]


def _node_call(x, a0, a1, w1a, w1b, w2, w3, b1t, b2t, b3t, gt, bt):
    n, d = x.shape
    de = a0.shape[1]
    bn = 2000
    grid = (n // bn,)
    blk = lambda i: (i, 0)
    zro = lambda i: (0, 0)
    return pl.pallas_call(
        _node_body,
        grid=grid,
        in_specs=[
            pl.BlockSpec((bn, d), blk),
            pl.BlockSpec((bn, de), blk),
            pl.BlockSpec((bn, de), blk),
            pl.BlockSpec((d, d), zro),
            pl.BlockSpec((de, d), zro),
            pl.BlockSpec((d, d), zro),
            pl.BlockSpec((d, d), zro),
            pl.BlockSpec((1, d), zro),
            pl.BlockSpec((1, d), zro),
            pl.BlockSpec((1, d), zro),
            pl.BlockSpec((1, d), zro),
            pl.BlockSpec((1, d), zro),
        ],
        out_specs=pl.BlockSpec((bn, d), blk),
        out_shape=jax.ShapeDtypeStruct((n, d), jnp.float32),
    )(x, a0, a1, w1a, w1b, w2, w3, b1t, b2t, b3t, gt, bt)


def kernel(x, edge_index, edge_attr,
           eW1, eb1, eW2, eb2, eW3, eb3, eg, ebt,
           nW1, nb1, nW2, nb2, nW3, nb3, ng, nbt):
    n, d = x.shape
    e, de = edge_attr.shape
    p = d // de
    assert e % (NW * SUB) == 0 and d == p * de and n % p == 0
    npad = ((n + NS * 8 - 1) // (NS * 8)) * NS * 8

    ii = edge_index[0].astype(jnp.int32)
    jj = edge_index[1].astype(jnp.int32)

    eye = jnp.eye(p, dtype=jnp.float32)
    w1a_sp = jnp.kron(eye, eW1[:d])
    w1b_sp = jnp.kron(eye, eW1[d:2 * d])

    x8 = x.reshape(n // p, p * d)
    xa_p, xb_p = _pre_call(x8, w1a_sp, w1b_sp)
    xa = xa_p.reshape(n, de)
    xb = xb_p.reshape(n, de)

    g_t = _gather_call(xa, xb, ii, jj, e, de)

    ea_t = edge_attr.T
    ea2_t = _edge_call(g_t, ea_t,
                       eW1[2 * d:].T, eW2.T, eW3.T,
                       eb1[:, None], eb2[:, None], eb3[:, None],
                       eg[:, None], ebt[:, None])
    edge_attr2 = ea2_t.T

    agg2 = _scatter_call(ea2_t, jj,
                         jnp.zeros((npad, de), jnp.float32), npad, e, de)

    x2 = _node_call(x, agg2[0], agg2[1],
                    nW1[:d], nW1[d:], nW2, nW3,
                    nb1[None], nb2[None], nb3[None], ng[None], nbt[None])
    return (x2, edge_attr2)


# speedup vs baseline: 1.0373x; 1.0373x over previous
import functools

import jax
import jax.numpy as jnp
from jax import lax
from jax.experimental import pallas as pl
from jax.experimental.pallas import tpu as pltpu
from jax.experimental.pallas import tpu_sc as plsc

NC = 2
NS = 16
NW = NC * NS
SUB = 80
NRING = 5

_EPS = 1e-5


def _pre_body(x_ref, wa_ref, wb_ref, xa_ref, xb_ref):
    xx = x_ref[...]
    xa_ref[...] = jnp.dot(xx, wa_ref[...], preferred_element_type=jnp.float32)
    xb_ref[...] = jnp.dot(xx, wb_ref[...], preferred_element_type=jnp.float32)


def _pre_call(x8, wa_sp, wb_sp):
    n8, dp = x8.shape
    bn = n8
    grid = (n8 // bn,)
    return pl.pallas_call(
        _pre_body,
        grid=grid,
        in_specs=[
            pl.BlockSpec((bn, dp), lambda i: (i, 0)),
            pl.BlockSpec((dp, 128), lambda i: (0, 0)),
            pl.BlockSpec((dp, 128), lambda i: (0, 0)),
        ],
        out_specs=[
            pl.BlockSpec((bn, 128), lambda i: (i, 0)),
            pl.BlockSpec((bn, 128), lambda i: (i, 0)),
        ],
        out_shape=[
            jax.ShapeDtypeStruct((n8, 128), jnp.float32),
            jax.ShapeDtypeStruct((n8, 128), jnp.float32),
        ],
    )(x8, wa_sp, wb_sp)


def _gather_call(xa, xb, ii, jj, e, de):
    ew = e // NW
    ch = ew // SUB
    assert ch % NRING == 0
    mesh = plsc.VectorSubcoreMesh(core_axis_name="c", subcore_axis_name="s")

    @functools.partial(
        pl.kernel,
        out_type=jax.ShapeDtypeStruct((de, e), jnp.float32),
        mesh=mesh,
        scratch_types=[
            pltpu.VMEM((ew,), jnp.int32),
            pltpu.VMEM((ew,), jnp.int32),
            pltpu.VMEM((NRING, SUB, de), jnp.float32),
            pltpu.VMEM((NRING, SUB, de), jnp.float32),
            pltpu.VMEM((NRING, de, SUB), jnp.float32),
            [pltpu.SemaphoreType.DMA] * NRING,
            [pltpu.SemaphoreType.DMA] * NRING,
        ],
        compiler_params=pltpu.CompilerParams(use_tc_tiling_on_sc=False, needs_layout_passes=False),
    )
    def gather_kernel(xa_hbm, xb_hbm, ii_hbm, jj_hbm, g_hbm,
                      ii_v, jj_v, ra_v, rb_v, st_v, sem_g, sem_s):
        cid = lax.axis_index("c")
        sid = lax.axis_index("s")
        wid = sid * NC + cid
        pltpu.sync_copy(ii_hbm.at[pl.ds(wid * ew, ew)], ii_v)
        pltpu.sync_copy(jj_hbm.at[pl.ds(wid * ew, ew)], jj_v)
        dummy = g_hbm.at[:, pl.ds(0, SUB)]
        dummy2 = xa_hbm.at[pl.ds(0, SUB)]
        lane = lax.iota(jnp.int32, de)

        def fire(k, u):
            idx = pl.multiple_of(k * SUB, 8)
            pltpu.async_copy(xa_hbm.at[ii_v.at[pl.ds(idx, SUB)]],
                             ra_v.at[u], sem_g[u])
            pltpu.async_copy(xb_hbm.at[jj_v.at[pl.ds(idx, SUB)]],
                             rb_v.at[u], sem_g[u])

        def drain_gather(u):
            pltpu.make_async_copy(dummy2, ra_v.at[u], sem_g[u]).wait()
            pltpu.make_async_copy(dummy2, rb_v.at[u], sem_g[u]).wait()

        fire(0, 0)
        fire(1, 1)
        fire(2, 2)

        def super_step(s5, carry):
            for u in range(NRING):
                k = s5 * NRING + u
                u3 = (u + 3) % NRING
                @pl.when(k >= 2)
                def _():
                    pltpu.make_async_copy(dummy, st_v.at[u3], sem_s[u3]).wait()

                @pl.when(k + 3 < ch)
                def _():
                    fire(k + 3, u3)

                drain_gather(u)

                def add_row(r, c2):
                    plsc.store_scatter(
                        st_v.at[u], [lane, jnp.full((de,), r, jnp.int32)],
                        ra_v[u, r, :] + rb_v[u, r, :])
                    return c2

                lax.fori_loop(0, SUB, add_row, 0, unroll=8)
                col0 = pl.multiple_of((wid * ch + k) * SUB, 8)
                pltpu.async_copy(
                    st_v.at[u], g_hbm.at[:, pl.ds(col0, SUB)], sem_s[u])
            return carry

        lax.fori_loop(0, ch // NRING, super_step, 0)
        for u in range(NRING - 2, NRING):
            pltpu.make_async_copy(dummy, st_v.at[u], sem_s[u]).wait()

    return gather_kernel(xa, xb, ii, jj)


def _edge_body(g_ref, ea_ref, w1c_ref, w2_ref, w3_ref,
               b1_ref, b2_ref, b3_ref, gam_ref, bet_ref, out_ref):
    ea = ea_ref[...]
    z = (jnp.dot(w1c_ref[...], ea, preferred_element_type=jnp.float32)
         + g_ref[...] + b1_ref[...])
    h = z * jax.nn.sigmoid(z)
    z2 = jnp.dot(w2_ref[...], h, preferred_element_type=jnp.float32) + b2_ref[...]
    h2 = z2 * jax.nn.sigmoid(z2)
    o = jnp.dot(w3_ref[...], h2, preferred_element_type=jnp.float32) + b3_ref[...]
    m = jnp.mean(o, axis=0, keepdims=True)
    dlt = o - m
    v = jnp.mean(dlt * dlt, axis=0, keepdims=True)
    ln = dlt * lax.rsqrt(v + _EPS) * gam_ref[...] + bet_ref[...]
    out_ref[...] = ea + ln


def _edge_call(g_t, ea_t, w1cT, w2T, w3T, b1c, b2c, b3c, gc, bc):
    de, e = g_t.shape
    be = 16000
    grid = (e // be,)
    blk = lambda i: (0, i)
    zro = lambda i: (0, 0)
    return pl.pallas_call(
        _edge_body,
        grid=grid,
        in_specs=[
            pl.BlockSpec((de, be), blk),
            pl.BlockSpec((de, be), blk),
            pl.BlockSpec((de, de), zro),
            pl.BlockSpec((de, de), zro),
            pl.BlockSpec((de, de), zro),
            pl.BlockSpec((de, 1), zro),
            pl.BlockSpec((de, 1), zro),
            pl.BlockSpec((de, 1), zro),
            pl.BlockSpec((de, 1), zro),
            pl.BlockSpec((de, 1), zro),
        ],
        out_specs=pl.BlockSpec((de, be), blk),
        out_shape=jax.ShapeDtypeStruct((de, e), jnp.float32),
    )(g_t, ea_t, w1cT, w2T, w3T, b1c, b2c, b3c, gc, bc)


def _scatter_call(ea2_flat, jj, zeros_nde, npad, e, de):
    ew = e // NW
    ch = ew // SUB
    assert ch % NRING == 0
    zr = npad // NS
    mesh = plsc.VectorSubcoreMesh(core_axis_name="c", subcore_axis_name="s")

    @functools.partial(
        pl.kernel,
        out_type=jax.ShapeDtypeStruct((NC, npad, de), jnp.float32),
        mesh=mesh,
        scratch_types=[
            pltpu.VMEM((ew,), jnp.int32),
            pltpu.VMEM((NRING, de, 5 * SUB), jnp.float32),
            pltpu.VMEM((5, SUB, de), jnp.float32),
            pltpu.VMEM_SHARED((npad, de), jnp.float32),
            [pltpu.SemaphoreType.DMA] * NRING,
            pltpu.SemaphoreType.DMA,
        ],
        compiler_params=pltpu.CompilerParams(use_tc_tiling_on_sc=False, needs_layout_passes=False),
    )
    def scatter_kernel(ea2_hbm, jj_hbm, z_hbm, agg_hbm,
                       jj_v, rows_v, st_v, acc_sh, sem_l, sem_sc):
        cid = lax.axis_index("c")
        sid = lax.axis_index("s")
        wid = sid * NC + cid
        lw = 5 * SUB
        nl = ch // 5
        pltpu.sync_copy(z_hbm.at[pl.ds(sid * zr, zr)],
                        acc_sh.at[pl.ds(sid * zr, zr)])
        plsc.subcore_barrier()
        pltpu.sync_copy(jj_hbm.at[pl.ds(wid * ew, ew)], jj_v)
        dummy = ea2_hbm.at[:, pl.ds(0, 5 * SUB)]
        lane = lax.iota(jnp.int32, de)

        def fire(l, u):
            col0 = pl.multiple_of((wid * nl + l) * lw, 8)
            pltpu.async_copy(ea2_hbm.at[:, pl.ds(col0, lw)],
                             rows_v.at[u], sem_l[u])

        fire(0, 0)
        fire(1, 1)

        def super_step(s5, carry):
            for u in range(NRING):
                l = s5 * NRING + u
                u2 = (u + 2) % NRING

                @pl.when(l + 2 < nl)
                def _():
                    fire(l + 2, u2)

                pltpu.make_async_copy(dummy, rows_v.at[u], sem_l[u]).wait()
                descs = []
                for s in range(5):
                    k = l * 5 + s

                    def stage_row(r, c2):
                        st_v[s, r, :] = plsc.load_gather(
                            rows_v.at[u],
                            [lane, jnp.full((de,), s * SUB + r, jnp.int32)])
                        return c2

                    lax.fori_loop(0, SUB, stage_row, 0, unroll=8)
                    idx = pl.multiple_of(k * SUB, 8)
                    descs.append(pltpu.async_copy(
                        st_v.at[s], acc_sh.at[jj_v.at[pl.ds(idx, SUB)]],
                        sem_sc, add=True))
                for dsc in descs:
                    dsc.wait()
            return carry

        lax.fori_loop(0, nl // NRING, super_step, 0)
        plsc.subcore_barrier()
        pltpu.sync_copy(acc_sh.at[pl.ds(sid * zr, zr)],
                        agg_hbm.at[cid].at[pl.ds(sid * zr, zr)])

    return scatter_kernel(ea2_flat, jj, zeros_nde)


def _node_body(x_ref, a0_ref, a1_ref, w1a_ref, w1b_ref, w2_ref, w3_ref,
               b1_ref, b2_ref, b3_ref, gam_ref, bet_ref, out_ref):
    xx = x_ref[...]
    agg = a0_ref[...] + a1_ref[...]
    z = (jnp.dot(xx, w1a_ref[...], preferred_element_type=jnp.float32)
         + jnp.dot(agg, w1b_ref[...], preferred_element_type=jnp.float32)
         + b1_ref[...])
    h = z * jax.nn.sigmoid(z)
    z2 = jnp.dot(h, w2_ref[...], preferred_element_type=jnp.float32) + b2_ref[...]
    h2 = z2 * jax.nn.sigmoid(z2)
    o = jnp.dot(h2, w3_ref[...], preferred_element_type=jnp.float32) + b3_ref[...]
    m = jnp.mean(o, axis=-1, keepdims=True)
    dlt = o - m
    v = jnp.mean(dlt * dlt, axis=-1, keepdims=True)
    out_ref[...] = xx + dlt * lax.rsqrt(v + _EPS) * gam_ref[...] + bet_ref[...]


def _node_call(x, a0, a1, w1a, w1b, w2, w3, b1t, b2t, b3t, gt, bt):
    n, d = x.shape
    de = a0.shape[1]
    bn = 2000
    grid = (n // bn,)
    blk = lambda i: (i, 0)
    zro = lambda i: (0, 0)
    return pl.pallas_call(
        _node_body,
        grid=grid,
        in_specs=[
            pl.BlockSpec((bn, d), blk),
            pl.BlockSpec((bn, de), blk),
            pl.BlockSpec((bn, de), blk),
            pl.BlockSpec((d, d), zro),
            pl.BlockSpec((de, d), zro),
            pl.BlockSpec((d, d), zro),
            pl.BlockSpec((d, d), zro),
            pl.BlockSpec((1, d), zro),
            pl.BlockSpec((1, d), zro),
            pl.BlockSpec((1, d), zro),
            pl.BlockSpec((1, d), zro),
            pl.BlockSpec((1, d), zro),
        ],
        out_specs=pl.BlockSpec((bn, d), blk),
        out_shape=jax.ShapeDtypeStruct((n, d), jnp.float32),
    )(x, a0, a1, w1a, w1b, w2, w3, b1t, b2t, b3t, gt, bt)


def kernel(x, edge_index, edge_attr,
           eW1, eb1, eW2, eb2, eW3, eb3, eg, ebt,
           nW1, nb1, nW2, nb2, nW3, nb3, ng, nbt):
    n, d = x.shape
    e, de = edge_attr.shape
    p = d // de
    assert e % (NW * SUB) == 0 and d == p * de and n % p == 0
    npad = ((n + NS * 8 - 1) // (NS * 8)) * NS * 8

    ii = edge_index[0].astype(jnp.int32)
    jj = edge_index[1].astype(jnp.int32)

    eye = jnp.eye(p, dtype=jnp.float32)
    w1a_sp = jnp.kron(eye, eW1[:d])
    w1b_sp = jnp.kron(eye, eW1[d:2 * d])

    x8 = x.reshape(n // p, p * d)
    xa_p, xb_p = _pre_call(x8, w1a_sp, w1b_sp)
    xa = xa_p.reshape(n, de)
    xb = xb_p.reshape(n, de)

    g_t = _gather_call(xa, xb, ii, jj, e, de)

    ea_t = edge_attr.T
    ea2_t = _edge_call(g_t, ea_t,
                       eW1[2 * d:].T, eW2.T, eW3.T,
                       eb1[:, None], eb2[:, None], eb3[:, None],
                       eg[:, None], ebt[:, None])
    edge_attr2 = ea2_t.T

    agg2 = _scatter_call(ea2_t, jj,
                         jnp.zeros((npad, de), jnp.float32), npad, e, de)

    x2 = _node_call(x, agg2[0], agg2[1],
                    nW1[:d], nW1[d:], nW2, nW3,
                    nb1[None], nb2[None], nb3[None], ng[None], nbt[None])
    return (x2, edge_attr2)


# speedup vs baseline: 1.0384x; 1.0010x over previous
import functools

import jax
import jax.numpy as jnp
from jax import lax
from jax.experimental import pallas as pl
from jax.experimental.pallas import tpu as pltpu
from jax.experimental.pallas import tpu_sc as plsc

NC = 2
NS = 16
NW = NC * NS
SUB = 80
NRING = 5

_EPS = 1e-5


def _pre_body(x_ref, wa_ref, wb_ref, xa_ref, xb_ref):
    xx = x_ref[...]
    xa_ref[...] = jnp.dot(xx, wa_ref[...], preferred_element_type=jnp.float32)
    xb_ref[...] = jnp.dot(xx, wb_ref[...], preferred_element_type=jnp.float32)


def _pre_call(x8, wa_sp, wb_sp):
    n8, dp = x8.shape
    bn = n8
    grid = (n8 // bn,)
    return pl.pallas_call(
        _pre_body,
        grid=grid,
        in_specs=[
            pl.BlockSpec((bn, dp), lambda i: (i, 0)),
            pl.BlockSpec((dp, 128), lambda i: (0, 0)),
            pl.BlockSpec((dp, 128), lambda i: (0, 0)),
        ],
        out_specs=[
            pl.BlockSpec((bn, 128), lambda i: (i, 0)),
            pl.BlockSpec((bn, 128), lambda i: (i, 0)),
        ],
        out_shape=[
            jax.ShapeDtypeStruct((n8, 128), jnp.float32),
            jax.ShapeDtypeStruct((n8, 128), jnp.float32),
        ],
    )(x8, wa_sp, wb_sp)


def _gather_call(xa, xb, ii, jj, e, de):
    ew = e // NW
    ch = ew // SUB
    assert ch % NRING == 0
    mesh = plsc.VectorSubcoreMesh(core_axis_name="c", subcore_axis_name="s")

    @functools.partial(
        pl.kernel,
        out_type=jax.ShapeDtypeStruct((de, e), jnp.float32),
        mesh=mesh,
        scratch_types=[
            pltpu.VMEM((ew,), jnp.int32),
            pltpu.VMEM((ew,), jnp.int32),
            pltpu.VMEM((NRING, SUB, de), jnp.float32),
            pltpu.VMEM((NRING, SUB, de), jnp.float32),
            pltpu.VMEM((NRING, de, SUB), jnp.float32),
            [pltpu.SemaphoreType.DMA] * NRING,
            [pltpu.SemaphoreType.DMA] * NRING,
        ],
        compiler_params=pltpu.CompilerParams(use_tc_tiling_on_sc=False, needs_layout_passes=False),
    )
    def gather_kernel(xa_hbm, xb_hbm, ii_hbm, jj_hbm, g_hbm,
                      ii_v, jj_v, ra_v, rb_v, st_v, sem_g, sem_s):
        cid = lax.axis_index("c")
        sid = lax.axis_index("s")
        wid = sid * NC + cid
        pltpu.sync_copy(ii_hbm.at[pl.ds(wid * ew, ew)], ii_v)
        pltpu.sync_copy(jj_hbm.at[pl.ds(wid * ew, ew)], jj_v)
        dummy = g_hbm.at[:, pl.ds(0, SUB)]
        dummy2 = xa_hbm.at[pl.ds(0, SUB)]
        lane = lax.iota(jnp.int32, de)

        def fire(k, u):
            idx = pl.multiple_of(k * SUB, 8)
            pltpu.async_copy(xa_hbm.at[ii_v.at[pl.ds(idx, SUB)]],
                             ra_v.at[u], sem_g[u])
            pltpu.async_copy(xb_hbm.at[jj_v.at[pl.ds(idx, SUB)]],
                             rb_v.at[u], sem_g[u])

        def drain_gather(u):
            pltpu.make_async_copy(dummy2, ra_v.at[u], sem_g[u]).wait()
            pltpu.make_async_copy(dummy2, rb_v.at[u], sem_g[u]).wait()

        fire(0, 0)
        fire(1, 1)
        fire(2, 2)

        def super_step(s5, carry):
            for u in range(NRING):
                k = s5 * NRING + u
                u3 = (u + 3) % NRING
                @pl.when(k >= 2)
                def _():
                    pltpu.make_async_copy(dummy, st_v.at[u3], sem_s[u3]).wait()

                @pl.when(k + 3 < ch)
                def _():
                    fire(k + 3, u3)

                drain_gather(u)

                def add_row(r, c2):
                    plsc.store_scatter(
                        st_v.at[u], [lane, jnp.full((de,), r, jnp.int32)],
                        ra_v[u, r, :] + rb_v[u, r, :])
                    return c2

                lax.fori_loop(0, SUB, add_row, 0, unroll=8)
                col0 = pl.multiple_of((wid * ch + k) * SUB, 8)
                pltpu.async_copy(
                    st_v.at[u], g_hbm.at[:, pl.ds(col0, SUB)], sem_s[u])
            return carry

        lax.fori_loop(0, ch // NRING, super_step, 0)
        for u in range(NRING - 2, NRING):
            pltpu.make_async_copy(dummy, st_v.at[u], sem_s[u]).wait()

    return gather_kernel(xa, xb, ii, jj)


def _edge_body(g_ref, ea_ref, w1c_ref, w2_ref, w3_ref,
               b1_ref, b2_ref, b3_ref, gam_ref, bet_ref, out_ref):
    ea = ea_ref[...]
    z = (jnp.dot(w1c_ref[...], ea, preferred_element_type=jnp.float32)
         + g_ref[...] + b1_ref[...])
    h = z * jax.nn.sigmoid(z)
    z2 = jnp.dot(w2_ref[...], h, preferred_element_type=jnp.float32) + b2_ref[...]
    h2 = z2 * jax.nn.sigmoid(z2)
    o = jnp.dot(w3_ref[...], h2, preferred_element_type=jnp.float32) + b3_ref[...]
    m = jnp.mean(o, axis=0, keepdims=True)
    dlt = o - m
    v = jnp.mean(dlt * dlt, axis=0, keepdims=True)
    ln = dlt * lax.rsqrt(v + _EPS) * gam_ref[...] + bet_ref[...]
    out_ref[...] = ea + ln


def _edge_call(g_t, ea_t, w1cT, w2T, w3T, b1c, b2c, b3c, gc, bc):
    de, e = g_t.shape
    be = 16000
    grid = (e // be,)
    blk = lambda i: (0, i)
    zro = lambda i: (0, 0)
    return pl.pallas_call(
        _edge_body,
        grid=grid,
        in_specs=[
            pl.BlockSpec((de, be), blk),
            pl.BlockSpec((de, be), blk),
            pl.BlockSpec((de, de), zro),
            pl.BlockSpec((de, de), zro),
            pl.BlockSpec((de, de), zro),
            pl.BlockSpec((de, 1), zro),
            pl.BlockSpec((de, 1), zro),
            pl.BlockSpec((de, 1), zro),
            pl.BlockSpec((de, 1), zro),
            pl.BlockSpec((de, 1), zro),
        ],
        out_specs=pl.BlockSpec((de, be), blk),
        out_shape=jax.ShapeDtypeStruct((de, e), jnp.float32),
    )(g_t, ea_t, w1cT, w2T, w3T, b1c, b2c, b3c, gc, bc)


def _scatter_call(ea2_flat, jj, zeros_nde, npad, e, de):
    ew = e // NW
    ch = ew // SUB
    assert ch % NRING == 0
    zr = npad // NS
    mesh = plsc.VectorSubcoreMesh(core_axis_name="c", subcore_axis_name="s")

    @functools.partial(
        pl.kernel,
        out_type=jax.ShapeDtypeStruct((NC, npad, de), jnp.float32),
        mesh=mesh,
        scratch_types=[
            pltpu.VMEM((ew,), jnp.int32),
            pltpu.VMEM((NRING, de, 5 * SUB), jnp.float32),
            pltpu.VMEM((5, SUB, de), jnp.float32),
            pltpu.VMEM_SHARED((npad, de), jnp.float32),
            [pltpu.SemaphoreType.DMA] * NRING,
            pltpu.SemaphoreType.DMA,
        ],
        compiler_params=pltpu.CompilerParams(use_tc_tiling_on_sc=False, needs_layout_passes=False),
    )
    def scatter_kernel(ea2_hbm, jj_hbm, z_hbm, agg_hbm,
                       jj_v, rows_v, st_v, acc_sh, sem_l, sem_sc):
        cid = lax.axis_index("c")
        sid = lax.axis_index("s")
        wid = sid * NC + cid
        lw = 5 * SUB
        nl = ch // 5
        pltpu.sync_copy(z_hbm.at[pl.ds(sid * zr, zr)],
                        acc_sh.at[pl.ds(sid * zr, zr)])
        plsc.subcore_barrier()
        pltpu.sync_copy(jj_hbm.at[pl.ds(wid * ew, ew)], jj_v)
        dummy = ea2_hbm.at[:, pl.ds(0, 5 * SUB)]
        lane = lax.iota(jnp.int32, de)

        def fire(l, u):
            col0 = pl.multiple_of((wid * nl + l) * lw, 8)
            pltpu.async_copy(ea2_hbm.at[:, pl.ds(col0, lw)],
                             rows_v.at[u], sem_l[u])

        fire(0, 0)
        fire(1, 1)

        def super_step(s5, carry):
            for u in range(NRING):
                l = s5 * NRING + u
                u2 = (u + 2) % NRING

                @pl.when(l + 2 < nl)
                def _():
                    fire(l + 2, u2)

                pltpu.make_async_copy(dummy, rows_v.at[u], sem_l[u]).wait()
                descs = []
                for s in range(5):
                    k = l * 5 + s

                    def stage_row(r, c2):
                        st_v[s, r, :] = plsc.load_gather(
                            rows_v.at[u],
                            [lane, jnp.full((de,), s * SUB + r, jnp.int32)])
                        return c2

                    lax.fori_loop(0, SUB, stage_row, 0, unroll=8)
                    idx = pl.multiple_of(k * SUB, 8)
                    descs.append(pltpu.async_copy(
                        st_v.at[s], acc_sh.at[jj_v.at[pl.ds(idx, SUB)]],
                        sem_sc, add=True))
                for dsc in descs:
                    dsc.wait()
            return carry

        lax.fori_loop(0, nl // NRING, super_step, 0)
        plsc.subcore_barrier()
        pltpu.sync_copy(acc_sh.at[pl.ds(sid * zr, zr)],
                        agg_hbm.at[cid].at[pl.ds(sid * zr, zr)])

    return scatter_kernel(ea2_flat, jj, zeros_nde)


def _node_body(x_ref, a0_ref, a1_ref, w1a_ref, w1b_ref, w2_ref, w3_ref,
               b1_ref, b2_ref, b3_ref, gam_ref, bet_ref, out_ref):
    xx = x_ref[...]
    agg = a0_ref[...] + a1_ref[...]
    z = (jnp.dot(xx, w1a_ref[...], preferred_element_type=jnp.float32)
         + jnp.dot(agg, w1b_ref[...], preferred_element_type=jnp.float32)
         + b1_ref[...])
    h = z * jax.nn.sigmoid(z)
    z2 = jnp.dot(h, w2_ref[...], preferred_element_type=jnp.float32) + b2_ref[...]
    h2 = z2 * jax.nn.sigmoid(z2)
    o = jnp.dot(h2, w3_ref[...], preferred_element_type=jnp.float32) + b3_ref[...]
    m = jnp.mean(o, axis=-1, keepdims=True)
    dlt = o - m
    v = jnp.mean(dlt * dlt, axis=-1, keepdims=True)
    out_ref[...] = xx + dlt * lax.rsqrt(v + _EPS) * gam_ref[...] + bet_ref[...]


def _node_call(x, a0, a1, w1a, w1b, w2, w3, b1t, b2t, b3t, gt, bt):
    n, d = x.shape
    de = a0.shape[1]
    bn = 2000
    grid = (n // bn,)
    blk = lambda i: (i, 0)
    zro = lambda i: (0, 0)
    return pl.pallas_call(
        _node_body,
        grid=grid,
        in_specs=[
            pl.BlockSpec((bn, d), blk),
            pl.BlockSpec((bn, de), blk),
            pl.BlockSpec((bn, de), blk),
            pl.BlockSpec((d, d), zro),
            pl.BlockSpec((de, d), zro),
            pl.BlockSpec((d, d), zro),
            pl.BlockSpec((d, d), zro),
            pl.BlockSpec((1, d), zro),
            pl.BlockSpec((1, d), zro),
            pl.BlockSpec((1, d), zro),
            pl.BlockSpec((1, d), zro),
            pl.BlockSpec((1, d), zro),
        ],
        out_specs=pl.BlockSpec((bn, d), blk),
        out_shape=jax.ShapeDtypeStruct((n, d), jnp.float32),
    )(x, a0, a1, w1a, w1b, w2, w3, b1t, b2t, b3t, gt, bt)


def kernel(x, edge_index, edge_attr,
           eW1, eb1, eW2, eb2, eW3, eb3, eg, ebt,
           nW1, nb1, nW2, nb2, nW3, nb3, ng, nbt):
    n, d = x.shape
    e, de = edge_attr.shape
    p = d // de
    assert e % (NW * SUB) == 0 and d == p * de and n % p == 0
    npad = ((n + NS * 8 - 1) // (NS * 8)) * NS * 8

    ii = edge_index[0].astype(jnp.int32)
    jj = edge_index[1].astype(jnp.int32)

    eye = jnp.eye(p, dtype=jnp.float32)
    w1a_sp = jnp.kron(eye, eW1[:d])
    w1b_sp = jnp.kron(eye, eW1[d:2 * d])

    ea_t = edge_attr.T
    x8 = x.reshape(n // p, p * d)
    xa_p, xb_p = _pre_call(x8, w1a_sp, w1b_sp)
    xa = xa_p.reshape(n, de)
    xb = xb_p.reshape(n, de)

    g_t = _gather_call(xa, xb, ii, jj, e, de)

    ea2_t = _edge_call(g_t, ea_t,
                       eW1[2 * d:].T, eW2.T, eW3.T,
                       eb1[:, None], eb2[:, None], eb3[:, None],
                       eg[:, None], ebt[:, None])
    edge_attr2 = ea2_t.T

    agg2 = _scatter_call(ea2_t, jj,
                         jnp.zeros((npad, de), jnp.float32), npad, e, de)

    x2 = _node_call(x, agg2[0], agg2[1],
                    nW1[:d], nW1[d:], nW2, nW3,
                    nb1[None], nb2[None], nb3[None], ng[None], nbt[None])
    return (x2, edge_attr2)
